# Initial kernel scaffold; baseline (speedup 1.0000x reference)
import functools

import jax
import jax.numpy as jnp
from jax import lax
from jax.experimental import pallas as pl
from jax.experimental.pallas import tpu as pltpu
from jax.experimental.pallas import tpu_sc as plsc

_USER = 25000
_ITEM = 25000
_N = _USER + _ITEM
_DH = 32
_E = 800000
_NC, _NS = 2, 16
_CHUNK = 128
_CHUNKS_PER_TILE = 391
_EPT = _CHUNK * _CHUNKS_PER_TILE
_E_PAD = _EPT * _NS
_ROWS_PER_TILE = _N // _NS
_WB = 625
_WB_STEPS = _ROWS_PER_TILE // _WB


@functools.partial(
    pl.kernel,
    out_type=[
        jax.ShapeDtypeStruct((2 * _N, _DH), jnp.float32),
        jax.ShapeDtypeStruct((2 * _N, _DH), jnp.float32),
        jax.ShapeDtypeStruct((2 * _N, _DH), jnp.float32),
        jax.ShapeDtypeStruct((2 * _N, _DH), jnp.float32),
    ],
    mesh=plsc.VectorSubcoreMesh(
        core_axis_name="c", subcore_axis_name="s",
        num_cores=_NC, num_subcores=_NS,
    ),
    scratch_types=[
        pltpu.VMEM((_CHUNK,), jnp.int32),
        pltpu.VMEM((_CHUNK,), jnp.int32),
        pltpu.VMEM((_CHUNK,), jnp.float32),
        pltpu.VMEM((_CHUNK, _DH), jnp.float32),
        pltpu.VMEM((_WB, _DH), jnp.float32),
        pltpu.VMEM((_WB, _DH), jnp.float32),
        pltpu.VMEM((_WB, _DH), jnp.float32),
        pltpu.VMEM((_WB, _DH), jnp.float32),
        pltpu.VMEM_SHARED((_N, _DH), jnp.float32),
        pltpu.SemaphoreType.DMA,
    ],
)
def _sc_prop(ego0, srcp, dstp, gainp, t1, t2, e3, mean_out,
             src_v, dst_v, gain_v, rows_v, b1, b2, b3, zero_v, acc, sem):
    cc = lax.axis_index("c")
    s = lax.axis_index("s")
    half = cc * _N
    r0 = s * _ROWS_PER_TILE
    e0 = s * _EPT

    def _zinit(i, carry):
        zero_v[i, pl.ds(0, 16)] = jnp.zeros((16,), jnp.float32)
        zero_v[i, pl.ds(16, 16)] = jnp.zeros((16,), jnp.float32)
        return carry
    lax.fori_loop(0, _WB, _zinit, 0)

    def run_layer(tin, write_fn):
        def _zs(i, carry):
            pltpu.sync_copy(zero_v, acc.at[pl.ds(r0 + i * _WB, _WB)])
            return carry
        lax.fori_loop(0, _WB_STEPS, _zs, 0)
        plsc.subcore_barrier()

        def _chunk(c, carry):
            base = e0 + c * _CHUNK
            pltpu.sync_copy(srcp.at[pl.ds(base, _CHUNK)], src_v)
            pltpu.sync_copy(dstp.at[pl.ds(base, _CHUNK)], dst_v)
            pltpu.sync_copy(gainp.at[pl.ds(base, _CHUNK)], gain_v)

            def _off(j, cy):
                src_v[pl.ds(j * 16, 16)] = src_v[pl.ds(j * 16, 16)] + half
                return cy
            lax.fori_loop(0, _CHUNK // 16, _off, 0)

            pltpu.async_copy(tin.at[src_v], rows_v, sem).wait()

            def _scale(e, cy):
                g = gain_v[e]
                rows_v[e, pl.ds(0, 16)] = rows_v[e, pl.ds(0, 16)] * g
                rows_v[e, pl.ds(16, 16)] = rows_v[e, pl.ds(16, 16)] * g
                return cy
            lax.fori_loop(0, _CHUNK, _scale, 0)

            pltpu.sync_copy(rows_v, acc.at[dst_v], add=True)
            return carry
        lax.fori_loop(0, _CHUNKS_PER_TILE, _chunk, 0)
        plsc.subcore_barrier()

        write_fn()
        plsc.subcore_barrier()

    def wb_plain(tout):
        def _w(i, carry):
            rr = r0 + i * _WB
            pltpu.sync_copy(acc.at[pl.ds(rr, _WB)], b3)
            pltpu.sync_copy(b3, tout.at[pl.ds(half + rr, _WB)])
            return carry
        lax.fori_loop(0, _WB_STEPS, _w, 0)

    def wb_final():
        def _w(i, carry):
            rr = r0 + i * _WB
            pltpu.sync_copy(t1.at[pl.ds(half + rr, _WB)], b1)
            pltpu.sync_copy(t2.at[pl.ds(half + rr, _WB)], b2)
            pltpu.sync_copy(acc.at[pl.ds(rr, _WB)], b3)
            pltpu.sync_copy(b3, e3.at[pl.ds(half + rr, _WB)])

            def _m(r, cy):
                x0 = (b1[r, pl.ds(0, 16)] + b2[r, pl.ds(0, 16)]) * 2.0 \
                    + b3[r, pl.ds(0, 16)]
                b1[r, pl.ds(0, 16)] = x0 * 0.2
                x1 = (b1[r, pl.ds(16, 16)] + b2[r, pl.ds(16, 16)]) * 2.0 \
                    + b3[r, pl.ds(16, 16)]
                b1[r, pl.ds(16, 16)] = x1 * 0.2
                return cy
            lax.fori_loop(0, _WB, _m, 0)
            pltpu.sync_copy(b1, mean_out.at[pl.ds(half + rr, _WB)])
            return carry
        lax.fori_loop(0, _WB_STEPS, _w, 0)

    run_layer(ego0, lambda: wb_plain(t1))
    run_layer(t1, lambda: wb_plain(t2))
    run_layer(t2, wb_final)


def kernel(user_emb, item_emb, adj_values, edge_index):
    ego = jnp.concatenate([user_emb, item_emb], axis=0)
    ego0 = jnp.concatenate([ego[:, :_DH], ego[:, _DH:]], axis=0)
    src = edge_index[0].astype(jnp.int32)
    dst = edge_index[1].astype(jnp.int32)
    gain = adj_values.astype(jnp.float32)
    pad = _E_PAD - _E
    pidx = (jnp.arange(pad, dtype=jnp.int32) * 61) % _N
    srcp = jnp.concatenate([src, pidx])
    dstp = jnp.concatenate([dst, pidx])
    gainp = jnp.concatenate([gain, jnp.zeros((pad,), jnp.float32)])

    t1, t2, e3, mean = _sc_prop(ego0, srcp, dstp, gainp)

    mean_full = jnp.concatenate([mean[:_N], mean[_N:]], axis=1)
    neg = jnp.concatenate([e3[_USER:_N], e3[_N + _USER:]], axis=1)
    return (mean_full[:_USER], mean_full[_USER:], neg)


# speedup vs baseline: 3.3836x; 3.3836x over previous
import functools

import jax
import jax.numpy as jnp
from jax import lax
from jax.experimental import pallas as pl
from jax.experimental.pallas import tpu as pltpu
from jax.experimental.pallas import tpu_sc as plsc

_USER = 25000
_ITEM = 25000
_N = _USER + _ITEM
_NP = 51200
_DH = 32
_E = 800000
_NC, _NS = 2, 16
_CHUNK = 128
_CHUNKS_PER_TILE = 391
_EPT = _CHUNK * _CHUNKS_PER_TILE
_E_PAD = _EPT * _NS
_ROWS_PER_TILE = _NP // _NS
_WB = 160
_WB_STEPS = _ROWS_PER_TILE // _WB


@functools.partial(
    pl.kernel,
    out_type=[
        jax.ShapeDtypeStruct((2 * _NP, _DH), jnp.float32),
        jax.ShapeDtypeStruct((2 * _NP, _DH), jnp.float32),
        jax.ShapeDtypeStruct((2 * _NP, _DH), jnp.float32),
        jax.ShapeDtypeStruct((2 * _NP, _DH), jnp.float32),
    ],
    mesh=plsc.VectorSubcoreMesh(
        core_axis_name="c", subcore_axis_name="s",
        num_cores=_NC, num_subcores=_NS,
    ),
    compiler_params=pltpu.CompilerParams(use_tc_tiling_on_sc=False),
    scratch_types=[
        pltpu.VMEM((_CHUNK,), jnp.int32),
        pltpu.VMEM((_CHUNK,), jnp.int32),
        pltpu.VMEM((_CHUNK,), jnp.float32),
        pltpu.VMEM((_CHUNK, _DH), jnp.float32),
        pltpu.VMEM((_WB, _DH), jnp.float32),
        pltpu.VMEM((_WB, _DH), jnp.float32),
        pltpu.VMEM((_WB, _DH), jnp.float32),
        pltpu.VMEM_SHARED((_NP, _DH), jnp.float32),
        pltpu.SemaphoreType.DMA,
    ],
)
def _sc_prop(ego0, srcp, dstp, gainp, t1, t2, e3, mean_out,
             src_v, dst_v, gain_v, rows_v, b1, b2, b3, acc, sem):
    cc = lax.axis_index("c")
    s = lax.axis_index("s")
    half = pl.multiple_of(cc * _NP, 8)
    r0 = pl.multiple_of(s * _ROWS_PER_TILE, 8)
    e0 = s * _EPT

    def _zinit(i, carry):
        b1[i, pl.ds(0, 16)] = jnp.zeros((16,), jnp.float32)
        b1[i, pl.ds(16, 16)] = jnp.zeros((16,), jnp.float32)
        return carry
    lax.fori_loop(0, _WB, _zinit, 0)

    def run_layer(tin, write_fn):
        def _zs(i, carry):
            pltpu.sync_copy(b1, acc.at[pl.ds(r0 + i * _WB, _WB)])
            return carry
        lax.fori_loop(0, _WB_STEPS, _zs, 0)
        plsc.subcore_barrier()

        def _chunk(c, carry):
            base = e0 + c * _CHUNK
            pltpu.sync_copy(srcp.at[pl.ds(base, _CHUNK)], src_v)
            pltpu.sync_copy(dstp.at[pl.ds(base, _CHUNK)], dst_v)
            pltpu.sync_copy(gainp.at[pl.ds(base, _CHUNK)], gain_v)

            def _off(j, cy):
                src_v[pl.ds(j * 16, 16)] = src_v[pl.ds(j * 16, 16)] + half
                return cy
            lax.fori_loop(0, _CHUNK // 16, _off, 0)

            pltpu.async_copy(tin.at[src_v], rows_v, sem).wait()

            def _scale(grp, cy):
                g16 = gain_v[pl.ds(grp * 16, 16)]
                e_base = grp * 16
                for j in range(16):
                    g = g16[j]
                    e = e_base + j
                    rows_v[e, pl.ds(0, 16)] = rows_v[e, pl.ds(0, 16)] * g
                    rows_v[e, pl.ds(16, 16)] = rows_v[e, pl.ds(16, 16)] * g
                return cy
            lax.fori_loop(0, _CHUNK // 16, _scale, 0)

            pltpu.sync_copy(rows_v, acc.at[dst_v], add=True)
            return carry
        lax.fori_loop(0, _CHUNKS_PER_TILE, _chunk, 0)
        plsc.subcore_barrier()

        write_fn()
        plsc.subcore_barrier()

    def wb_plain(tout):
        def _w(i, carry):
            rr = r0 + i * _WB
            pltpu.sync_copy(acc.at[pl.ds(rr, _WB)], b3)
            pltpu.sync_copy(b3, tout.at[pl.ds(half + rr, _WB)])
            return carry
        lax.fori_loop(0, _WB_STEPS, _w, 0)

    def wb_final():
        def _w(i, carry):
            rr = r0 + i * _WB
            pltpu.sync_copy(t1.at[pl.ds(half + rr, _WB)], b1)
            pltpu.sync_copy(t2.at[pl.ds(half + rr, _WB)], b2)
            pltpu.sync_copy(acc.at[pl.ds(rr, _WB)], b3)
            pltpu.sync_copy(b3, e3.at[pl.ds(half + rr, _WB)])

            def _m(r, cy):
                x0 = (b1[r, pl.ds(0, 16)] + b2[r, pl.ds(0, 16)]) * 2.0 \
                    + b3[r, pl.ds(0, 16)]
                b1[r, pl.ds(0, 16)] = x0 * 0.2
                x1 = (b1[r, pl.ds(16, 16)] + b2[r, pl.ds(16, 16)]) * 2.0 \
                    + b3[r, pl.ds(16, 16)]
                b1[r, pl.ds(16, 16)] = x1 * 0.2
                return cy
            lax.fori_loop(0, _WB, _m, 0)
            pltpu.sync_copy(b1, mean_out.at[pl.ds(half + rr, _WB)])
            return carry
        lax.fori_loop(0, _WB_STEPS, _w, 0)

    run_layer(ego0, lambda: wb_plain(t1))
    run_layer(t1, lambda: wb_plain(t2))
    run_layer(t2, wb_final)


def kernel(user_emb, item_emb, adj_values, edge_index):
    ego = jnp.concatenate([user_emb, item_emb], axis=0)
    rpad = jnp.zeros((_NP - _N, _DH), jnp.float32)
    ego0 = jnp.concatenate([ego[:, :_DH], rpad, ego[:, _DH:], rpad], axis=0)
    src = edge_index[0].astype(jnp.int32)
    dst = edge_index[1].astype(jnp.int32)
    gain = adj_values.astype(jnp.float32)
    pad = _E_PAD - _E
    pidx = (jnp.arange(pad, dtype=jnp.int32) * 61) % _N
    srcp = jnp.concatenate([src, pidx])
    dstp = jnp.concatenate([dst, pidx])
    gainp = jnp.concatenate([gain, jnp.zeros((pad,), jnp.float32)])

    t1, t2, e3, mean = _sc_prop(ego0, srcp, dstp, gainp)

    mean_full = jnp.concatenate([mean[:_N], mean[_NP:_NP + _N]], axis=1)
    neg = jnp.concatenate([e3[_USER:_N], e3[_NP + _USER:_NP + _N]], axis=1)
    return (mean_full[:_USER], mean_full[_USER:], neg)


# speedup vs baseline: 6.3179x; 1.8672x over previous
import functools

import jax
import jax.numpy as jnp
from jax import lax
from jax.experimental import pallas as pl
from jax.experimental.pallas import tpu as pltpu
from jax.experimental.pallas import tpu_sc as plsc

_USER = 25000
_ITEM = 25000
_N = _USER + _ITEM
_NP = 51200
_DH = 32
_E = 800000
_NC, _NS = 2, 16
_CHUNK = 128
_NCH = 392
_EPT = _CHUNK * _NCH
_E_PAD = _EPT * _NS
_ROWS_PER_TILE = _NP // _NS
_WB = 80
_WB_STEPS = _ROWS_PER_TILE // _WB


@functools.partial(
    pl.kernel,
    out_type=[
        jax.ShapeDtypeStruct((2 * _NP, _DH), jnp.float32),
        jax.ShapeDtypeStruct((2 * _NP, _DH), jnp.float32),
        jax.ShapeDtypeStruct((2 * _NP, _DH), jnp.float32),
        jax.ShapeDtypeStruct((2 * _NP, _DH), jnp.float32),
    ],
    mesh=plsc.VectorSubcoreMesh(
        core_axis_name="c", subcore_axis_name="s",
        num_cores=_NC, num_subcores=_NS,
    ),
    compiler_params=pltpu.CompilerParams(
        use_tc_tiling_on_sc=False, needs_layout_passes=False),
    scratch_types=[
        pltpu.VMEM((3, _CHUNK), jnp.int32),
        pltpu.VMEM((3, _CHUNK), jnp.int32),
        pltpu.VMEM((_CHUNK, _DH), jnp.float32),
        pltpu.VMEM((_CHUNK, _DH), jnp.float32),
        pltpu.VMEM((_WB, _DH), jnp.float32),
        pltpu.VMEM((_WB, _DH), jnp.float32),
        pltpu.VMEM((_WB, _DH), jnp.float32),
        pltpu.VMEM_SHARED((_NP, _DH), jnp.float32),
        pltpu.SemaphoreType.DMA,
        pltpu.SemaphoreType.DMA,
        pltpu.SemaphoreType.DMA,
        pltpu.SemaphoreType.DMA,
        pltpu.SemaphoreType.DMA,
        pltpu.SemaphoreType.DMA,
    ],
)
def _sc_prop(ego0, meta_h, t1, t2, e3, mean_out,
             meta_a, meta_b, rows_a, rows_b, b1, b2, b3, acc,
             sem_ma, sem_mb, sem_ga, sem_gb, sem_sa, sem_sb):
    cc = lax.axis_index("c")
    s = lax.axis_index("s")
    half = pl.multiple_of(cc * _NP, 8)
    r0 = pl.multiple_of(s * _ROWS_PER_TILE, 8)
    ch0 = s * _NCH

    def _zinit(i, carry):
        b1[i, pl.ds(0, 16)] = jnp.zeros((16,), jnp.float32)
        b1[i, pl.ds(16, 16)] = jnp.zeros((16,), jnp.float32)
        return carry
    lax.fori_loop(0, _WB, _zinit, 0)

    def meta_start(idx, mbuf, sem):
        pltpu.async_copy(meta_h.at[cc, idx], mbuf, sem)

    def meta_wait(mbuf, sem):
        pltpu.make_async_copy(meta_h.at[cc, 0], mbuf, sem).wait()

    def gather_start(tin, mbuf, rbuf, sem):
        pltpu.async_copy(tin.at[mbuf.at[0]], rbuf, sem)

    def gather_wait(tin, mbuf, rbuf, sem):
        pltpu.make_async_copy(tin.at[mbuf.at[0]], rbuf, sem).wait()

    def scatter_start(mbuf, rbuf, sem):
        pltpu.make_async_copy(rbuf, acc.at[mbuf.at[1]], sem).start(add=True)

    def scatter_wait(mbuf, rbuf, sem):
        pltpu.make_async_copy(rbuf, acc.at[mbuf.at[1]], sem).wait()

    def scale(mbuf, rbuf):
        def _grp(grp, cy):
            g16 = plsc.bitcast(mbuf[2, pl.ds(grp * 16, 16)], jnp.float32)
            e_base = grp * 16
            for j in range(16):
                g = g16[j]
                e = e_base + j
                rbuf[e, pl.ds(0, 16)] = rbuf[e, pl.ds(0, 16)] * g
                rbuf[e, pl.ds(16, 16)] = rbuf[e, pl.ds(16, 16)] * g
            return cy
        lax.fori_loop(0, _CHUNK // 16, _grp, 0)

    def run_layer(tin, write_fn):
        def _zs(i, carry):
            pltpu.sync_copy(b1, acc.at[pl.ds(r0 + i * _WB, _WB)])
            return carry
        lax.fori_loop(0, _WB_STEPS, _zs, 0)
        plsc.subcore_barrier()

        meta_start(ch0, meta_a, sem_ma)
        meta_start(ch0 + 1, meta_b, sem_mb)
        meta_wait(meta_a, sem_ma)
        gather_start(tin, meta_a, rows_a, sem_ga)

        def _pair(k, carry):
            c0 = ch0 + 2 * k
            meta_wait(meta_b, sem_mb)
            gather_wait(tin, meta_a, rows_a, sem_ga)
            gather_start(tin, meta_b, rows_b, sem_gb)
            scale(meta_a, rows_a)
            scatter_start(meta_a, rows_a, sem_sa)
            gather_wait(tin, meta_b, rows_b, sem_gb)
            scale(meta_b, rows_b)
            scatter_start(meta_b, rows_b, sem_sb)
            scatter_wait(meta_a, rows_a, sem_sa)
            meta_start(c0 + 2, meta_a, sem_ma)
            meta_wait(meta_a, sem_ma)
            gather_start(tin, meta_a, rows_a, sem_ga)
            scatter_wait(meta_b, rows_b, sem_sb)
            meta_start(c0 + 3, meta_b, sem_mb)
            return carry
        lax.fori_loop(0, _NCH // 2 - 1, _pair, 0)

        meta_wait(meta_b, sem_mb)
        gather_wait(tin, meta_a, rows_a, sem_ga)
        gather_start(tin, meta_b, rows_b, sem_gb)
        scale(meta_a, rows_a)
        scatter_start(meta_a, rows_a, sem_sa)
        gather_wait(tin, meta_b, rows_b, sem_gb)
        scale(meta_b, rows_b)
        scatter_start(meta_b, rows_b, sem_sb)
        scatter_wait(meta_a, rows_a, sem_sa)
        scatter_wait(meta_b, rows_b, sem_sb)
        plsc.subcore_barrier()

        write_fn()
        plsc.subcore_barrier()

    def wb_plain(tout):
        def _w(i, carry):
            rr = r0 + i * _WB
            pltpu.sync_copy(acc.at[pl.ds(rr, _WB)], b3)
            pltpu.sync_copy(b3, tout.at[pl.ds(half + rr, _WB)])
            return carry
        lax.fori_loop(0, _WB_STEPS, _w, 0)

    def wb_final():
        def _w(i, carry):
            rr = r0 + i * _WB
            pltpu.sync_copy(t1.at[pl.ds(half + rr, _WB)], b1)
            pltpu.sync_copy(t2.at[pl.ds(half + rr, _WB)], b2)
            pltpu.sync_copy(acc.at[pl.ds(rr, _WB)], b3)
            pltpu.sync_copy(b3, e3.at[pl.ds(half + rr, _WB)])

            def _m(r, cy):
                x0 = (b1[r, pl.ds(0, 16)] + b2[r, pl.ds(0, 16)]) * 2.0 \
                    + b3[r, pl.ds(0, 16)]
                b1[r, pl.ds(0, 16)] = x0 * 0.2
                x1 = (b1[r, pl.ds(16, 16)] + b2[r, pl.ds(16, 16)]) * 2.0 \
                    + b3[r, pl.ds(16, 16)]
                b1[r, pl.ds(16, 16)] = x1 * 0.2
                return cy
            lax.fori_loop(0, _WB, _m, 0)
            pltpu.sync_copy(b1, mean_out.at[pl.ds(half + rr, _WB)])
            return carry
        lax.fori_loop(0, _WB_STEPS, _w, 0)

    run_layer(ego0, lambda: wb_plain(t1))
    run_layer(t1, lambda: wb_plain(t2))
    run_layer(t2, wb_final)


def kernel(user_emb, item_emb, adj_values, edge_index):
    ego = jnp.concatenate([user_emb, item_emb], axis=0)
    rpad = jnp.zeros((_NP - _N, _DH), jnp.float32)
    ego0 = jnp.concatenate([ego[:, :_DH], rpad, ego[:, _DH:], rpad], axis=0)
    src = edge_index[0].astype(jnp.int32)
    dst = edge_index[1].astype(jnp.int32)
    gain = adj_values.astype(jnp.float32)
    pad = _E_PAD - _E
    pidx = (jnp.arange(pad, dtype=jnp.int32) * 61) % _N
    srcp = jnp.concatenate([src, pidx])
    dstp = jnp.concatenate([dst, pidx])
    gbits = lax.bitcast_convert_type(
        jnp.concatenate([gain, jnp.zeros((pad,), jnp.float32)]), jnp.int32)
    meta = jnp.stack(
        [jnp.stack([srcp, dstp, gbits]), jnp.stack([srcp + _NP, dstp, gbits])])
    meta = meta.reshape(_NC, 3, _NS * _NCH, _CHUNK).transpose(0, 2, 1, 3)

    t1, t2, e3, mean = _sc_prop(ego0, meta, )

    mean_full = jnp.concatenate([mean[:_N], mean[_NP:_NP + _N]], axis=1)
    neg = jnp.concatenate([e3[_USER:_N], e3[_NP + _USER:_NP + _N]], axis=1)
    return (mean_full[:_USER], mean_full[_USER:], neg)


# speedup vs baseline: 7.0476x; 1.1155x over previous
import functools

import jax
import jax.numpy as jnp
from jax import lax
from jax.experimental import pallas as pl
from jax.experimental.pallas import tpu as pltpu
from jax.experimental.pallas import tpu_sc as plsc

_USER = 25000
_ITEM = 25000
_N = _USER + _ITEM
_NP = 51200
_DH = 32
_E = 800000
_NC, _NS = 2, 16
_CHUNK = 128
_NCH = 392
_NSETS = 4
_NRINGS = _NCH // _NSETS
_EPT = _CHUNK * _NCH
_E_PAD = _EPT * _NS
_RPT = _NP // _NS
_WB = 128
_WB_STEPS = _RPT // _WB


@functools.partial(
    pl.kernel,
    out_type=[
        jax.ShapeDtypeStruct((2 * _NP, _DH), jnp.float32),
        jax.ShapeDtypeStruct((2 * _NP, _DH), jnp.float32),
        jax.ShapeDtypeStruct((2 * _NP, _DH), jnp.float32),
        jax.ShapeDtypeStruct((2 * _NP, _DH), jnp.float32),
    ],
    mesh=plsc.VectorSubcoreMesh(
        core_axis_name="c", subcore_axis_name="s",
        num_cores=_NC, num_subcores=_NS,
    ),
    compiler_params=pltpu.CompilerParams(
        use_tc_tiling_on_sc=False, needs_layout_passes=False),
    scratch_types=(
        [pltpu.VMEM((3, _CHUNK), jnp.int32)] * _NSETS
        + [pltpu.VMEM((_CHUNK, _DH), jnp.float32)] * _NSETS
        + [pltpu.VMEM((_CHUNK,), jnp.int32)] * _NSETS
        + [pltpu.VMEM_SHARED((_NP, _DH), jnp.float32)]
        + [pltpu.SemaphoreType.DMA] * (3 * _NSETS)
    ),
)
def _sc_prop(ego0, meta_h, zeros_h, t1, t2, e3, mean_out, *scr):
    meta = scr[0:4]
    rows = scr[4:8]
    dstv = scr[8:12]
    acc = scr[12]
    sem_m = scr[13:17]
    sem_g = scr[17:21]
    sem_s = scr[21:25]

    cc = lax.axis_index("c")
    s = lax.axis_index("s")
    half = pl.multiple_of(cc * _NP, 8)
    r0 = pl.multiple_of(s * _RPT, 8)
    ch0 = s * _NCH

    def meta_start(idx, p):
        pltpu.async_copy(meta_h.at[cc, idx], meta[p], sem_m[p])

    def meta_wait(p):
        pltpu.make_async_copy(meta_h.at[cc, 0], meta[p], sem_m[p]).wait()

    def gather_start(tin, p):
        pltpu.async_copy(tin.at[meta[p].at[0]], rows[p], sem_g[p])

    def gather_wait(tin, p):
        pltpu.make_async_copy(tin.at[meta[p].at[0]], rows[p], sem_g[p]).wait()

    def scatter_start(p):
        pltpu.make_async_copy(rows[p], acc.at[dstv[p]], sem_s[p]).start(add=True)

    def scatter_wait(p):
        pltpu.make_async_copy(rows[p], acc.at[dstv[p]], sem_s[p]).wait()

    def dst_copy(p):
        for j in range(_CHUNK // 16):
            dstv[p][pl.ds(j * 16, 16)] = meta[p][1, pl.ds(j * 16, 16)]

    def scale(p):
        def _grp(grp, cy):
            g16 = plsc.bitcast(meta[p][2, pl.ds(grp * 16, 16)], jnp.float32)
            e_base = grp * 16
            for j in range(16):
                g = g16[j]
                e = e_base + j
                rows[p][e, pl.ds(0, 16)] = rows[p][e, pl.ds(0, 16)] * g
                rows[p][e, pl.ds(16, 16)] = rows[p][e, pl.ds(16, 16)] * g
            return cy
        lax.fori_loop(0, _CHUNK // 16, _grp, 0)

    def run_layer(tin, write_fn):
        pltpu.sync_copy(zeros_h, acc.at[pl.ds(r0, _RPT)])
        plsc.subcore_barrier()

        for p in range(_NSETS):
            meta_start(ch0 + p, p)
        meta_wait(0)
        gather_start(tin, 0)

        def _ring(k, carry):
            for p in range(_NSETS):
                c = ch0 + _NSETS * k + p
                r = (p + 1) % _NSETS
                gather_wait(tin, p)
                dst_copy(p)

                @pl.when(k < _NRINGS - 1)
                def _():
                    meta_start(c + _NSETS, p)

                scale(p)
                if p == _NSETS - 1:
                    scatter_wait(r)
                else:
                    @pl.when(k >= 1)
                    def _():
                        scatter_wait(r)
                if p == _NSETS - 1:
                    @pl.when(k < _NRINGS - 1)
                    def _():
                        meta_wait(r)
                        gather_start(tin, r)
                else:
                    meta_wait(r)
                    gather_start(tin, r)
                scatter_start(p)
            return carry
        lax.fori_loop(0, _NRINGS, _ring, 0)
        for p in range(1, _NSETS):
            scatter_wait(p)
        plsc.subcore_barrier()

        write_fn()
        plsc.subcore_barrier()

    def wb_plain(tout):
        pltpu.sync_copy(acc.at[pl.ds(r0, _RPT)], tout.at[pl.ds(half + r0, _RPT)])

    def wb_final():
        pltpu.sync_copy(acc.at[pl.ds(r0, _RPT)], e3.at[pl.ds(half + r0, _RPT)])
        b1, b2, b3 = rows[0], rows[1], rows[2]

        def _w(i, carry):
            rr = r0 + i * _WB
            pltpu.sync_copy(t1.at[pl.ds(half + rr, _WB)], b1)
            pltpu.sync_copy(t2.at[pl.ds(half + rr, _WB)], b2)
            pltpu.sync_copy(acc.at[pl.ds(rr, _WB)], b3)

            def _m(rI, cy):
                x0 = (b1[rI, pl.ds(0, 16)] + b2[rI, pl.ds(0, 16)]) * 2.0 \
                    + b3[rI, pl.ds(0, 16)]
                b1[rI, pl.ds(0, 16)] = x0 * 0.2
                x1 = (b1[rI, pl.ds(16, 16)] + b2[rI, pl.ds(16, 16)]) * 2.0 \
                    + b3[rI, pl.ds(16, 16)]
                b1[rI, pl.ds(16, 16)] = x1 * 0.2
                return cy
            lax.fori_loop(0, _WB, _m, 0)
            pltpu.sync_copy(b1, mean_out.at[pl.ds(half + rr, _WB)])
            return carry
        lax.fori_loop(0, _WB_STEPS, _w, 0)

    run_layer(ego0, lambda: wb_plain(t1))
    run_layer(t1, lambda: wb_plain(t2))
    run_layer(t2, wb_final)


def kernel(user_emb, item_emb, adj_values, edge_index):
    ego = jnp.concatenate([user_emb, item_emb], axis=0)
    rpad = jnp.zeros((_NP - _N, _DH), jnp.float32)
    ego0 = jnp.concatenate([ego[:, :_DH], rpad, ego[:, _DH:], rpad], axis=0)
    src = edge_index[0].astype(jnp.int32)
    dst = edge_index[1].astype(jnp.int32)
    gain = adj_values.astype(jnp.float32)
    pad = _E_PAD - _E
    pidx = (jnp.arange(pad, dtype=jnp.int32) * 61) % _N
    srcp = jnp.concatenate([src, pidx])
    dstp = jnp.concatenate([dst, pidx])
    gbits = lax.bitcast_convert_type(
        jnp.concatenate([gain, jnp.zeros((pad,), jnp.float32)]), jnp.int32)
    meta = jnp.stack(
        [jnp.stack([srcp, dstp, gbits]), jnp.stack([srcp + _NP, dstp, gbits])])
    meta = meta.reshape(_NC, 3, _NS * _NCH, _CHUNK).transpose(0, 2, 1, 3)
    zeros_h = jnp.zeros((_RPT, _DH), jnp.float32)

    t1, t2, e3, mean = _sc_prop(ego0, meta, zeros_h)

    mean_full = jnp.concatenate([mean[:_N], mean[_NP:_NP + _N]], axis=1)
    neg = jnp.concatenate([e3[_USER:_N], e3[_NP + _USER:_NP + _N]], axis=1)
    return (mean_full[:_USER], mean_full[_USER:], neg)


# speedup vs baseline: 8.9187x; 1.2655x over previous
import functools

import jax
import jax.numpy as jnp
from jax import lax
from jax.experimental import pallas as pl
from jax.experimental.pallas import tpu as pltpu
from jax.experimental.pallas import tpu_sc as plsc

_USER = 25000
_ITEM = 25000
_N = _USER + _ITEM
_NP = 51200
_DH = 32
_E = 800000
_NC, _NS = 2, 16
_CHUNK = 256
_NCH = 198
_NSETS = 3
_NRINGS = _NCH // _NSETS
_EPT = _CHUNK * _NCH
_E_PAD = _EPT * _NS
_RPT = _NP // _NS
_WB = 128
_WB_STEPS = _RPT // _WB


@functools.partial(
    pl.kernel,
    out_type=[
        jax.ShapeDtypeStruct((2 * _NP, _DH), jnp.float32),
        jax.ShapeDtypeStruct((2 * _NP, _DH), jnp.float32),
        jax.ShapeDtypeStruct((2 * _NP, _DH), jnp.float32),
        jax.ShapeDtypeStruct((2 * _NP, _DH), jnp.float32),
    ],
    mesh=plsc.VectorSubcoreMesh(
        core_axis_name="c", subcore_axis_name="s",
        num_cores=_NC, num_subcores=_NS,
    ),
    compiler_params=pltpu.CompilerParams(
        use_tc_tiling_on_sc=False, needs_layout_passes=False),
    scratch_types=(
        [pltpu.VMEM((3, _CHUNK), jnp.int32)] * _NSETS
        + [pltpu.VMEM((_CHUNK, _DH), jnp.float32)] * _NSETS
        + [pltpu.VMEM((_CHUNK,), jnp.int32)] * _NSETS
        + [pltpu.VMEM_SHARED((_NP, _DH), jnp.float32)]
        + [pltpu.SemaphoreType.DMA] * (3 * _NSETS)
    ),
)
def _sc_prop(ego0, meta_h, zeros_h, t1, t2, e3, mean_out, *scr):
    meta = scr[0:_NSETS]
    rows = scr[_NSETS:2 * _NSETS]
    dstv = scr[2 * _NSETS:3 * _NSETS]
    acc = scr[3 * _NSETS]
    sem_m = scr[3 * _NSETS + 1:4 * _NSETS + 1]
    sem_g = scr[4 * _NSETS + 1:5 * _NSETS + 1]
    sem_s = scr[5 * _NSETS + 1:6 * _NSETS + 1]

    cc = lax.axis_index("c")
    s = lax.axis_index("s")
    half = pl.multiple_of(cc * _NP, 8)
    r0 = pl.multiple_of(s * _RPT, 8)
    ch0 = s * _NCH

    def meta_start(idx, p):
        pltpu.async_copy(meta_h.at[cc, idx], meta[p], sem_m[p])

    def meta_wait(p):
        pltpu.make_async_copy(meta_h.at[cc, 0], meta[p], sem_m[p]).wait()

    def gather_start(tin, p):
        pltpu.async_copy(tin.at[meta[p].at[0]], rows[p], sem_g[p])

    def gather_wait(tin, p):
        pltpu.make_async_copy(tin.at[meta[p].at[0]], rows[p], sem_g[p]).wait()

    def scatter_start(p):
        pltpu.make_async_copy(rows[p], acc.at[dstv[p]], sem_s[p]).start(add=True)

    def scatter_wait(p):
        pltpu.make_async_copy(rows[p], acc.at[dstv[p]], sem_s[p]).wait()

    def dst_copy(p):
        for j in range(_CHUNK // 16):
            dstv[p][pl.ds(j * 16, 16)] = meta[p][1, pl.ds(j * 16, 16)]

    def scale(p):
        def _grp(grp, cy):
            g16 = plsc.bitcast(meta[p][2, pl.ds(grp * 16, 16)], jnp.float32)
            e_base = grp * 16
            for j in range(16):
                g = g16[j]
                e = e_base + j
                rows[p][e, pl.ds(0, 16)] = rows[p][e, pl.ds(0, 16)] * g
                rows[p][e, pl.ds(16, 16)] = rows[p][e, pl.ds(16, 16)] * g
            return cy
        lax.fori_loop(0, _CHUNK // 16, _grp, 0)

    def run_layer(tin, write_fn):
        pltpu.sync_copy(zeros_h, acc.at[pl.ds(r0, _RPT)])
        plsc.subcore_barrier()

        for p in range(_NSETS):
            meta_start(ch0 + p, p)
        meta_wait(0)
        gather_start(tin, 0)

        def _ring(k, carry):
            for p in range(_NSETS):
                c = ch0 + _NSETS * k + p
                r = (p + 1) % _NSETS
                gather_wait(tin, p)
                dst_copy(p)

                @pl.when(k < _NRINGS - 1)
                def _():
                    meta_start(c + _NSETS, p)

                scale(p)
                if p == _NSETS - 1:
                    scatter_wait(r)
                else:
                    @pl.when(k >= 1)
                    def _():
                        scatter_wait(r)
                if p == _NSETS - 1:
                    @pl.when(k < _NRINGS - 1)
                    def _():
                        meta_wait(r)
                        gather_start(tin, r)
                else:
                    meta_wait(r)
                    gather_start(tin, r)
                scatter_start(p)
            return carry
        lax.fori_loop(0, _NRINGS, _ring, 0)
        for p in range(1, _NSETS):
            scatter_wait(p)
        plsc.subcore_barrier()

        write_fn()
        plsc.subcore_barrier()

    def wb_plain(tout):
        pltpu.sync_copy(acc.at[pl.ds(r0, _RPT)], tout.at[pl.ds(half + r0, _RPT)])

    def wb_final():
        pltpu.sync_copy(acc.at[pl.ds(r0, _RPT)], e3.at[pl.ds(half + r0, _RPT)])
        b1, b2, b3 = rows[0], rows[1], rows[2]

        def _mean_step(rr, size):
            pltpu.sync_copy(t1.at[pl.ds(half + rr, size)],
                            b1.at[pl.ds(0, size)])
            pltpu.sync_copy(t2.at[pl.ds(half + rr, size)],
                            b2.at[pl.ds(0, size)])
            pltpu.sync_copy(acc.at[pl.ds(rr, size)], b3.at[pl.ds(0, size)])

            def _m(rI, cy):
                x0 = (b1[rI, pl.ds(0, 16)] + b2[rI, pl.ds(0, 16)]) * 2.0 \
                    + b3[rI, pl.ds(0, 16)]
                b1[rI, pl.ds(0, 16)] = x0 * 0.2
                x1 = (b1[rI, pl.ds(16, 16)] + b2[rI, pl.ds(16, 16)]) * 2.0 \
                    + b3[rI, pl.ds(16, 16)]
                b1[rI, pl.ds(16, 16)] = x1 * 0.2
                return cy
            lax.fori_loop(0, size, _m, 0)
            pltpu.sync_copy(b1.at[pl.ds(0, size)],
                            mean_out.at[pl.ds(half + rr, size)])

        def _w(i, carry):
            _mean_step(r0 + i * _CHUNK, _CHUNK)
            return carry
        lax.fori_loop(0, _RPT // _CHUNK, _w, 0)
        rem = _RPT % _CHUNK
        if rem:
            _mean_step(r0 + (_RPT // _CHUNK) * _CHUNK, rem)

    run_layer(ego0, lambda: wb_plain(t1))
    run_layer(t1, lambda: wb_plain(t2))
    run_layer(t2, wb_final)


def kernel(user_emb, item_emb, adj_values, edge_index):
    ego = jnp.concatenate([user_emb, item_emb], axis=0)
    rpad = jnp.zeros((_NP - _N, _DH), jnp.float32)
    ego0 = jnp.concatenate([ego[:, :_DH], rpad, ego[:, _DH:], rpad], axis=0)
    src = edge_index[0].astype(jnp.int32)
    dst = edge_index[1].astype(jnp.int32)
    gain = adj_values.astype(jnp.float32)
    pad = _E_PAD - _E
    pidx = (jnp.arange(pad, dtype=jnp.int32) * 61) % _N
    srcp = jnp.concatenate([src, pidx])
    dstp = jnp.concatenate([dst, pidx])
    gbits = lax.bitcast_convert_type(
        jnp.concatenate([gain, jnp.zeros((pad,), jnp.float32)]), jnp.int32)
    meta = jnp.stack(
        [jnp.stack([srcp, dstp, gbits]), jnp.stack([srcp + _NP, dstp, gbits])])
    meta = meta.reshape(_NC, 3, _NS * _NCH, _CHUNK).transpose(0, 2, 1, 3)
    zeros_h = jnp.zeros((_RPT, _DH), jnp.float32)

    t1, t2, e3, mean = _sc_prop(ego0, meta, zeros_h)

    mean_full = jnp.concatenate([mean[:_N], mean[_NP:_NP + _N]], axis=1)
    neg = jnp.concatenate([e3[_USER:_N], e3[_NP + _USER:_NP + _N]], axis=1)
    return (mean_full[:_USER], mean_full[_USER:], neg)


# speedup vs baseline: 9.7172x; 1.0895x over previous
import functools

import jax
import jax.numpy as jnp
from jax import lax
from jax.experimental import pallas as pl
from jax.experimental.pallas import tpu as pltpu
from jax.experimental.pallas import tpu_sc as plsc

_USER = 25000
_ITEM = 25000
_N = _USER + _ITEM
_NP = 51200
_DH = 32
_E = 800000
_NC, _NS = 2, 16
_CHUNK = 256
_NCH = 198
_NSETS = 3
_NRINGS = _NCH // _NSETS
_EPT = _CHUNK * _NCH
_E_PAD = _EPT * _NS
_RPT = _NP // _NS
_WB = 128
_WB_STEPS = _RPT // _WB


@functools.partial(
    pl.kernel,
    out_type=[
        jax.ShapeDtypeStruct((2 * _NP, _DH), jnp.float32),
        jax.ShapeDtypeStruct((2 * _NP, _DH), jnp.float32),
        jax.ShapeDtypeStruct((_NP, 2 * _DH), jnp.float32),
        jax.ShapeDtypeStruct((_NP, 2 * _DH), jnp.float32),
    ],
    mesh=plsc.VectorSubcoreMesh(
        core_axis_name="c", subcore_axis_name="s",
        num_cores=_NC, num_subcores=_NS,
    ),
    compiler_params=pltpu.CompilerParams(
        use_tc_tiling_on_sc=False, needs_layout_passes=False),
    scratch_types=(
        [pltpu.VMEM((3, _CHUNK), jnp.int32)] * _NSETS
        + [pltpu.VMEM((_CHUNK, _DH), jnp.float32)] * _NSETS
        + [pltpu.VMEM((_CHUNK,), jnp.int32)] * _NSETS
        + [pltpu.VMEM_SHARED((_NP, _DH), jnp.float32)]
        + [pltpu.SemaphoreType.DMA] * (3 * _NSETS)
    ),
)
def _sc_prop(ego0, meta_h, zeros_h, t1, t2, e3, mean_out, *scr):
    meta = scr[0:_NSETS]
    rows = scr[_NSETS:2 * _NSETS]
    dstv = scr[2 * _NSETS:3 * _NSETS]
    acc = scr[3 * _NSETS]
    sem_m = scr[3 * _NSETS + 1:4 * _NSETS + 1]
    sem_g = scr[4 * _NSETS + 1:5 * _NSETS + 1]
    sem_s = scr[5 * _NSETS + 1:6 * _NSETS + 1]

    cc = lax.axis_index("c")
    s = lax.axis_index("s")
    half = pl.multiple_of(cc * _NP, 8)
    r0 = pl.multiple_of(s * _RPT, 8)
    ch0 = s * _NCH

    def meta_start(idx, p):
        pltpu.async_copy(meta_h.at[idx], meta[p], sem_m[p])

    def meta_wait(p):
        pltpu.make_async_copy(meta_h.at[0], meta[p], sem_m[p]).wait()

    def src_offset(p):
        @pl.when(cc == 1)
        def _():
            for j in range(_CHUNK // 16):
                meta[p][0, pl.ds(j * 16, 16)] = \
                    meta[p][0, pl.ds(j * 16, 16)] + _NP

    def gather_start(tin, p):
        src_offset(p)
        pltpu.async_copy(tin.at[meta[p].at[0]], rows[p], sem_g[p])

    def gather_wait(tin, p):
        pltpu.make_async_copy(tin.at[meta[p].at[0]], rows[p], sem_g[p]).wait()

    def scatter_start(p):
        pltpu.make_async_copy(rows[p], acc.at[dstv[p]], sem_s[p]).start(add=True)

    def scatter_wait(p):
        pltpu.make_async_copy(rows[p], acc.at[dstv[p]], sem_s[p]).wait()

    def dst_copy(p):
        for j in range(_CHUNK // 16):
            dstv[p][pl.ds(j * 16, 16)] = meta[p][1, pl.ds(j * 16, 16)]

    def scale(p):
        def _grp(grp, cy):
            g16 = plsc.bitcast(meta[p][2, pl.ds(grp * 16, 16)], jnp.float32)
            e_base = grp * 16
            for j in range(16):
                g = g16[j]
                e = e_base + j
                rows[p][e, pl.ds(0, 16)] = rows[p][e, pl.ds(0, 16)] * g
                rows[p][e, pl.ds(16, 16)] = rows[p][e, pl.ds(16, 16)] * g
            return cy
        lax.fori_loop(0, _CHUNK // 16, _grp, 0)

    def run_layer(tin, write_fn):
        pltpu.sync_copy(zeros_h, acc.at[pl.ds(r0, _RPT)])
        plsc.subcore_barrier()

        for p in range(_NSETS):
            meta_start(ch0 + p, p)
        meta_wait(0)
        gather_start(tin, 0)

        def _ring(k, carry):
            for p in range(_NSETS):
                c = ch0 + _NSETS * k + p
                r = (p + 1) % _NSETS
                gather_wait(tin, p)
                dst_copy(p)

                @pl.when(k < _NRINGS - 1)
                def _():
                    meta_start(c + _NSETS, p)

                scale(p)
                if p == _NSETS - 1:
                    scatter_wait(r)
                else:
                    @pl.when(k >= 1)
                    def _():
                        scatter_wait(r)
                if p == _NSETS - 1:
                    @pl.when(k < _NRINGS - 1)
                    def _():
                        meta_wait(r)
                        gather_start(tin, r)
                else:
                    meta_wait(r)
                    gather_start(tin, r)
                scatter_start(p)
            return carry
        lax.fori_loop(0, _NRINGS, _ring, 0)
        for p in range(1, _NSETS):
            scatter_wait(p)
        plsc.subcore_barrier()

        write_fn()
        plsc.subcore_barrier()

    def wb_plain(tout):
        pltpu.sync_copy(acc.at[pl.ds(r0, _RPT)], tout.at[pl.ds(half + r0, _RPT)])

    def wb_final():
        col = cc * _DH
        pltpu.sync_copy(acc.at[pl.ds(r0, _RPT)],
                        e3.at[pl.ds(r0, _RPT), pl.ds(col, _DH)])
        b1, b2, b3 = rows[0], rows[1], rows[2]

        def _mean_step(rr, size):
            pltpu.sync_copy(t1.at[pl.ds(half + rr, size)],
                            b1.at[pl.ds(0, size)])
            pltpu.sync_copy(t2.at[pl.ds(half + rr, size)],
                            b2.at[pl.ds(0, size)])
            pltpu.sync_copy(acc.at[pl.ds(rr, size)], b3.at[pl.ds(0, size)])

            def _m(rI, cy):
                x0 = (b1[rI, pl.ds(0, 16)] + b2[rI, pl.ds(0, 16)]) * 2.0 \
                    + b3[rI, pl.ds(0, 16)]
                b1[rI, pl.ds(0, 16)] = x0 * 0.2
                x1 = (b1[rI, pl.ds(16, 16)] + b2[rI, pl.ds(16, 16)]) * 2.0 \
                    + b3[rI, pl.ds(16, 16)]
                b1[rI, pl.ds(16, 16)] = x1 * 0.2
                return cy
            lax.fori_loop(0, size, _m, 0)
            pltpu.sync_copy(b1.at[pl.ds(0, size)],
                            mean_out.at[pl.ds(rr, size), pl.ds(col, _DH)])

        def _w(i, carry):
            _mean_step(r0 + i * _CHUNK, _CHUNK)
            return carry
        lax.fori_loop(0, _RPT // _CHUNK, _w, 0)
        rem = _RPT % _CHUNK
        if rem:
            _mean_step(r0 + (_RPT // _CHUNK) * _CHUNK, rem)

    run_layer(ego0, lambda: wb_plain(t1))
    run_layer(t1, lambda: wb_plain(t2))
    run_layer(t2, wb_final)


def kernel(user_emb, item_emb, adj_values, edge_index):
    ego = jnp.concatenate([user_emb, item_emb], axis=0)
    rpad = jnp.zeros((_NP - _N, _DH), jnp.float32)
    ego0 = jnp.concatenate([ego[:, :_DH], rpad, ego[:, _DH:], rpad], axis=0)
    src = edge_index[0].astype(jnp.int32)
    dst = edge_index[1].astype(jnp.int32)
    gain = adj_values.astype(jnp.float32)
    pad = _E_PAD - _E
    pidx = (jnp.arange(pad, dtype=jnp.int32) * 61) % _N
    srcp = jnp.concatenate([src, pidx])
    dstp = jnp.concatenate([dst, pidx])
    gbits = lax.bitcast_convert_type(
        jnp.concatenate([gain, jnp.zeros((pad,), jnp.float32)]), jnp.int32)
    meta = jnp.stack([srcp.reshape(_NS * _NCH, _CHUNK),
                      dstp.reshape(_NS * _NCH, _CHUNK),
                      gbits.reshape(_NS * _NCH, _CHUNK)], axis=1)
    zeros_h = jnp.zeros((_RPT, _DH), jnp.float32)

    t1, t2, e3, mean = _sc_prop(ego0, meta, zeros_h)

    return (mean[:_USER], mean[_USER:_N], e3[_USER:_N])


# speedup vs baseline: 11.8571x; 1.2202x over previous
import functools

import jax
import jax.numpy as jnp
from jax import lax
from jax.experimental import pallas as pl
from jax.experimental.pallas import tpu as pltpu
from jax.experimental.pallas import tpu_sc as plsc

_USER = 25000
_ITEM = 25000
_N = _USER + _ITEM
_NP = 51200
_DH = 32
_E = 800000
_NC, _NS = 2, 16
_CHUNK = 256
_NCH = 198
_NSETS = 3
_NRINGS = _NCH // _NSETS
_EPT = _CHUNK * _NCH
_E_PAD = _EPT * _NS
_RPT = _NP // _NS
_WB = 128
_WB_STEPS = _RPT // _WB


@functools.partial(
    pl.kernel,
    out_type=[
        jax.ShapeDtypeStruct((2 * _NP, _DH), jnp.float32),
        jax.ShapeDtypeStruct((2 * _NP, _DH), jnp.float32),
        jax.ShapeDtypeStruct((_NP, 2 * _DH), jnp.float32),
        jax.ShapeDtypeStruct((_NP, 2 * _DH), jnp.float32),
    ],
    mesh=plsc.VectorSubcoreMesh(
        core_axis_name="c", subcore_axis_name="s",
        num_cores=_NC, num_subcores=_NS,
    ),
    compiler_params=pltpu.CompilerParams(
        use_tc_tiling_on_sc=False, needs_layout_passes=False),
    scratch_types=(
        [pltpu.VMEM((3, _CHUNK), jnp.int32)] * _NSETS
        + [pltpu.VMEM((_CHUNK, _DH), jnp.float32)] * _NSETS
        + [pltpu.VMEM((_CHUNK,), jnp.int32)] * _NSETS
        + [pltpu.VMEM_SHARED((_NP, _DH), jnp.float32)]
        + [pltpu.SemaphoreType.DMA] * (3 * _NSETS)
    ),
)
def _sc_prop(ego0, meta_h, zeros_h, t1, t2, e3, mean_out, *scr):
    meta = scr[0:_NSETS]
    rows = scr[_NSETS:2 * _NSETS]
    dstv = scr[2 * _NSETS:3 * _NSETS]
    acc = scr[3 * _NSETS]
    sem_m = scr[3 * _NSETS + 1:4 * _NSETS + 1]
    sem_g = scr[4 * _NSETS + 1:5 * _NSETS + 1]
    sem_s = scr[5 * _NSETS + 1:6 * _NSETS + 1]

    cc = lax.axis_index("c")
    s = lax.axis_index("s")
    half = pl.multiple_of(cc * _NP, 8)
    r0 = pl.multiple_of(s * _RPT, 8)
    ch0 = s * _NCH

    def meta_start(idx, p):
        pltpu.async_copy(meta_h.at[idx], meta[p], sem_m[p])

    def meta_wait(p):
        pltpu.make_async_copy(meta_h.at[0], meta[p], sem_m[p]).wait()

    def src_offset(p):
        @pl.when(cc == 1)
        def _():
            for j in range(_CHUNK // 16):
                meta[p][0, pl.ds(j * 16, 16)] = \
                    meta[p][0, pl.ds(j * 16, 16)] + _NP

    def gather_start(tin, p):
        src_offset(p)
        pltpu.async_copy(tin.at[meta[p].at[0]], rows[p], sem_g[p])

    def gather_wait(tin, p):
        pltpu.make_async_copy(tin.at[meta[p].at[0]], rows[p], sem_g[p]).wait()

    def scatter_start(p):
        pltpu.make_async_copy(rows[p], acc.at[dstv[p]], sem_s[p]).start(add=True)

    def scatter_wait(p):
        pltpu.make_async_copy(rows[p], acc.at[dstv[p]], sem_s[p]).wait()

    def dst_copy(p):
        for j in range(_CHUNK // 16):
            dstv[p][pl.ds(j * 16, 16)] = meta[p][1, pl.ds(j * 16, 16)]

    def scale(p):
        def _grp(grp, cy):
            g16 = plsc.bitcast(meta[p][2, pl.ds(grp * 16, 16)], jnp.float32)
            e_base = grp * 16
            for j in range(16):
                g = g16[j]
                e = e_base + j
                rows[p][e, pl.ds(0, 16)] = rows[p][e, pl.ds(0, 16)] * g
                rows[p][e, pl.ds(16, 16)] = rows[p][e, pl.ds(16, 16)] * g
            return cy
        lax.fori_loop(0, _CHUNK // 16, _grp, 0)

    def run_layer(tin, write_fn):
        pltpu.sync_copy(zeros_h, acc.at[pl.ds(r0, _RPT)])
        plsc.subcore_barrier()

        for p in range(_NSETS):
            meta_start(ch0 + p, p)
        meta_wait(0)
        gather_start(tin, 0)

        def _ring(k, carry):
            for p in range(_NSETS):
                c = ch0 + _NSETS * k + p
                r = (p + 1) % _NSETS
                gather_wait(tin, p)
                dst_copy(p)

                @pl.when(k < _NRINGS - 1)
                def _():
                    meta_start(c + _NSETS, p)

                if p == _NSETS - 1:
                    scatter_wait(r)
                else:
                    @pl.when(k >= 1)
                    def _():
                        scatter_wait(r)
                if p == _NSETS - 1:
                    @pl.when(k < _NRINGS - 1)
                    def _():
                        meta_wait(r)
                        gather_start(tin, r)
                else:
                    meta_wait(r)
                    gather_start(tin, r)
                scale(p)
                scatter_start(p)
            return carry
        lax.fori_loop(0, _NRINGS, _ring, 0)
        for p in range(1, _NSETS):
            scatter_wait(p)
        plsc.subcore_barrier()

        write_fn()
        plsc.subcore_barrier()

    def wb_plain(tout):
        pltpu.sync_copy(acc.at[pl.ds(r0, _RPT)], tout.at[pl.ds(half + r0, _RPT)])

    def wb_final():
        col = cc * _DH
        pltpu.sync_copy(acc.at[pl.ds(r0, _RPT)],
                        e3.at[pl.ds(r0, _RPT), pl.ds(col, _DH)])
        b1, b2, b3 = rows[0], rows[1], rows[2]

        def _mean_step(rr, size):
            pltpu.sync_copy(t1.at[pl.ds(half + rr, size)],
                            b1.at[pl.ds(0, size)])
            pltpu.sync_copy(t2.at[pl.ds(half + rr, size)],
                            b2.at[pl.ds(0, size)])
            pltpu.sync_copy(acc.at[pl.ds(rr, size)], b3.at[pl.ds(0, size)])

            def _m(rI, cy):
                x0 = (b1[rI, pl.ds(0, 16)] + b2[rI, pl.ds(0, 16)]) * 2.0 \
                    + b3[rI, pl.ds(0, 16)]
                b1[rI, pl.ds(0, 16)] = x0 * 0.2
                x1 = (b1[rI, pl.ds(16, 16)] + b2[rI, pl.ds(16, 16)]) * 2.0 \
                    + b3[rI, pl.ds(16, 16)]
                b1[rI, pl.ds(16, 16)] = x1 * 0.2
                return cy
            lax.fori_loop(0, size, _m, 0)
            pltpu.sync_copy(b1.at[pl.ds(0, size)],
                            mean_out.at[pl.ds(rr, size), pl.ds(col, _DH)])

        def _w(i, carry):
            _mean_step(r0 + i * _CHUNK, _CHUNK)
            return carry
        lax.fori_loop(0, _RPT // _CHUNK, _w, 0)
        rem = _RPT % _CHUNK
        if rem:
            _mean_step(r0 + (_RPT // _CHUNK) * _CHUNK, rem)

    run_layer(ego0, lambda: wb_plain(t1))
    run_layer(t1, lambda: wb_plain(t2))
    run_layer(t2, wb_final)


def kernel(user_emb, item_emb, adj_values, edge_index):
    ego = jnp.concatenate([user_emb, item_emb], axis=0)
    rpad = jnp.zeros((_NP - _N, _DH), jnp.float32)
    ego0 = jnp.concatenate([ego[:, :_DH], rpad, ego[:, _DH:], rpad], axis=0)
    src = edge_index[0].astype(jnp.int32)
    dst = edge_index[1].astype(jnp.int32)
    gain = adj_values.astype(jnp.float32)
    pad = _E_PAD - _E
    pidx = (jnp.arange(pad, dtype=jnp.int32) * 61) % _N
    srcp = jnp.concatenate([src, pidx])
    dstp = jnp.concatenate([dst, pidx])
    gbits = lax.bitcast_convert_type(
        jnp.concatenate([gain, jnp.zeros((pad,), jnp.float32)]), jnp.int32)
    meta = jnp.stack([srcp.reshape(_NS * _NCH, _CHUNK),
                      dstp.reshape(_NS * _NCH, _CHUNK),
                      gbits.reshape(_NS * _NCH, _CHUNK)], axis=1)
    zeros_h = jnp.zeros((_RPT, _DH), jnp.float32)

    t1, t2, e3, mean = _sc_prop(ego0, meta, zeros_h)

    return (mean[:_USER], mean[_USER:_N], e3[_USER:_N])


# speedup vs baseline: 11.9063x; 1.0041x over previous
import functools

import jax
import jax.numpy as jnp
from jax import lax
from jax.experimental import pallas as pl
from jax.experimental.pallas import tpu as pltpu
from jax.experimental.pallas import tpu_sc as plsc

_USER = 25000
_ITEM = 25000
_N = _USER + _ITEM
_NP = 51200
_DH = 32
_E = 800000
_NC, _NS = 2, 16
_CHUNK = 256
_NCH = 198
_NSETS = 3
_NRINGS = _NCH // _NSETS
_EPT = _CHUNK * _NCH
_E_PAD = _EPT * _NS
_RPT = _NP // _NS
_WB = 128
_WB_STEPS = _RPT // _WB


@functools.partial(
    pl.kernel,
    out_type=[
        jax.ShapeDtypeStruct((2 * _NP, _DH), jnp.float32),
        jax.ShapeDtypeStruct((2 * _NP, _DH), jnp.float32),
        jax.ShapeDtypeStruct((_NP, 2 * _DH), jnp.float32),
        jax.ShapeDtypeStruct((_NP, 2 * _DH), jnp.float32),
    ],
    mesh=plsc.VectorSubcoreMesh(
        core_axis_name="c", subcore_axis_name="s",
        num_cores=_NC, num_subcores=_NS,
    ),
    compiler_params=pltpu.CompilerParams(
        use_tc_tiling_on_sc=False, needs_layout_passes=False),
    scratch_types=(
        [pltpu.VMEM((3, _CHUNK), jnp.int32)] * _NSETS
        + [pltpu.VMEM((_CHUNK, _DH), jnp.float32)] * _NSETS
        + [pltpu.VMEM((_CHUNK,), jnp.int32)] * _NSETS
        + [pltpu.VMEM_SHARED((_NP, _DH), jnp.float32)]
        + [pltpu.SemaphoreType.DMA] * (3 * _NSETS)
    ),
)
def _sc_prop(ego0, meta_h, zeros_h, t1, t2, e3, mean_out, *scr):
    meta = scr[0:_NSETS]
    rows = scr[_NSETS:2 * _NSETS]
    dstv = scr[2 * _NSETS:3 * _NSETS]
    acc = scr[3 * _NSETS]
    sem_m = scr[3 * _NSETS + 1:4 * _NSETS + 1]
    sem_g = scr[4 * _NSETS + 1:5 * _NSETS + 1]
    sem_s = scr[5 * _NSETS + 1:6 * _NSETS + 1]

    cc = lax.axis_index("c")
    s = lax.axis_index("s")
    half = pl.multiple_of(cc * _NP, 8)
    r0 = pl.multiple_of(s * _RPT, 8)
    ch0 = s * _NCH

    def meta_start(idx, p):
        pltpu.async_copy(meta_h.at[idx], meta[p], sem_m[p])

    def meta_wait(p):
        pltpu.make_async_copy(meta_h.at[0], meta[p], sem_m[p]).wait()

    def src_offset(p):
        @pl.when(cc == 1)
        def _():
            for j in range(_CHUNK // 16):
                meta[p][0, pl.ds(j * 16, 16)] = \
                    meta[p][0, pl.ds(j * 16, 16)] + _NP

    def gather_start(tin, p):
        src_offset(p)
        pltpu.async_copy(tin.at[meta[p].at[0]], rows[p], sem_g[p])

    def gather_wait(tin, p):
        pltpu.make_async_copy(tin.at[meta[p].at[0]], rows[p], sem_g[p]).wait()

    def scatter_start(p):
        pltpu.make_async_copy(rows[p], acc.at[dstv[p]], sem_s[p]).start(add=True)

    def scatter_wait(p):
        pltpu.make_async_copy(rows[p], acc.at[dstv[p]], sem_s[p]).wait()

    def dst_copy(p):
        for j in range(_CHUNK // 16):
            dstv[p][pl.ds(j * 16, 16)] = meta[p][1, pl.ds(j * 16, 16)]

    def scale(p):
        def _grp(grp, cy):
            g16 = plsc.bitcast(meta[p][2, pl.ds(grp * 16, 16)], jnp.float32)
            e_base = grp * 16
            for j in range(16):
                g = g16[j]
                e = e_base + j
                rows[p][e, pl.ds(0, 16)] = rows[p][e, pl.ds(0, 16)] * g
                rows[p][e, pl.ds(16, 16)] = rows[p][e, pl.ds(16, 16)] * g
            return cy
        lax.fori_loop(0, _CHUNK // 16, _grp, 0)

    def run_layer(tin, write_fn):
        pltpu.sync_copy(zeros_h, acc.at[pl.ds(r0, _RPT)])
        plsc.subcore_barrier()

        for p in range(_NSETS):
            meta_start(ch0 + p, p)
        meta_wait(0)
        gather_start(tin, 0)

        def _ring(k, carry):
            for p in range(_NSETS):
                c = ch0 + _NSETS * k + p
                r = (p + 1) % _NSETS
                gather_wait(tin, p)
                dst_copy(p)
                if p == _NSETS - 1:
                    scatter_wait(r)
                else:
                    @pl.when(k >= 1)
                    def _():
                        scatter_wait(r)
                if p == _NSETS - 1:
                    @pl.when(k < _NRINGS - 1)
                    def _():
                        meta_wait(r)
                        gather_start(tin, r)
                else:
                    meta_wait(r)
                    gather_start(tin, r)
                scale(p)
                scatter_start(p)
                @pl.when(k < _NRINGS - 1)
                def _():
                    meta_start(c + _NSETS, p)
            return carry
        lax.fori_loop(0, _NRINGS, _ring, 0)
        for p in range(1, _NSETS):
            scatter_wait(p)
        plsc.subcore_barrier()

        write_fn()
        plsc.subcore_barrier()

    def wb_plain(tout):
        pltpu.sync_copy(acc.at[pl.ds(r0, _RPT)], tout.at[pl.ds(half + r0, _RPT)])

    def wb_final():
        col = cc * _DH
        pltpu.sync_copy(acc.at[pl.ds(r0, _RPT)],
                        e3.at[pl.ds(r0, _RPT), pl.ds(col, _DH)])
        b1, b2, b3 = rows[0], rows[1], rows[2]

        def _mean_step(rr, size):
            pltpu.sync_copy(t1.at[pl.ds(half + rr, size)],
                            b1.at[pl.ds(0, size)])
            pltpu.sync_copy(t2.at[pl.ds(half + rr, size)],
                            b2.at[pl.ds(0, size)])
            pltpu.sync_copy(acc.at[pl.ds(rr, size)], b3.at[pl.ds(0, size)])

            def _m(rI, cy):
                x0 = (b1[rI, pl.ds(0, 16)] + b2[rI, pl.ds(0, 16)]) * 2.0 \
                    + b3[rI, pl.ds(0, 16)]
                b1[rI, pl.ds(0, 16)] = x0 * 0.2
                x1 = (b1[rI, pl.ds(16, 16)] + b2[rI, pl.ds(16, 16)]) * 2.0 \
                    + b3[rI, pl.ds(16, 16)]
                b1[rI, pl.ds(16, 16)] = x1 * 0.2
                return cy
            lax.fori_loop(0, size, _m, 0)
            pltpu.sync_copy(b1.at[pl.ds(0, size)],
                            mean_out.at[pl.ds(rr, size), pl.ds(col, _DH)])

        def _w(i, carry):
            _mean_step(r0 + i * _CHUNK, _CHUNK)
            return carry
        lax.fori_loop(0, _RPT // _CHUNK, _w, 0)
        rem = _RPT % _CHUNK
        if rem:
            _mean_step(r0 + (_RPT // _CHUNK) * _CHUNK, rem)

    run_layer(ego0, lambda: wb_plain(t1))
    run_layer(t1, lambda: wb_plain(t2))
    run_layer(t2, wb_final)


def kernel(user_emb, item_emb, adj_values, edge_index):
    ego = jnp.concatenate([user_emb, item_emb], axis=0)
    rpad = jnp.zeros((_NP - _N, _DH), jnp.float32)
    ego0 = jnp.concatenate([ego[:, :_DH], rpad, ego[:, _DH:], rpad], axis=0)
    src = edge_index[0].astype(jnp.int32)
    dst = edge_index[1].astype(jnp.int32)
    gain = adj_values.astype(jnp.float32)
    pad = _E_PAD - _E
    pidx = (jnp.arange(pad, dtype=jnp.int32) * 61) % _N
    srcp = jnp.concatenate([src, pidx])
    dstp = jnp.concatenate([dst, pidx])
    gbits = lax.bitcast_convert_type(
        jnp.concatenate([gain, jnp.zeros((pad,), jnp.float32)]), jnp.int32)
    meta = jnp.stack([srcp.reshape(_NS * _NCH, _CHUNK),
                      dstp.reshape(_NS * _NCH, _CHUNK),
                      gbits.reshape(_NS * _NCH, _CHUNK)], axis=1)
    zeros_h = jnp.zeros((_RPT, _DH), jnp.float32)

    t1, t2, e3, mean = _sc_prop(ego0, meta, zeros_h)

    return (mean[:_USER], mean[_USER:_N], e3[_USER:_N])
